# baseline (speedup 1.0000x reference)
import jax
import jax.numpy as jnp
from jax import lax
from jax.experimental import pallas as pl
from jax.experimental.pallas import tpu as pltpu
from jax.experimental.pallas import tpu_sc as plsc

N = 10000
E = 320000
D = 128
NPAD = 10240
NC = 2
NS = 16
NW = NC * NS

BLK = 512
GRID = NPAD // BLK

EPAD = 327680
EPW = EPAD // NW
EPT = EPAD // NS
CH = 128
SEG = 5120
NSEG = EPT // SEG
SEGC = SEG // CH
SEGP = SEGC // 2
RPT = NPAD // NS
RPW = NPAD // NW
CHA = 64


def _mesh():
    return plsc.VectorSubcoreMesh(
        core_axis_name="c", subcore_axis_name="s",
        num_cores=NC, num_subcores=NS)


def _stage_a_body(dst_hbm, perm_hbm, x_hbm, degp_hbm, xn_hbm,
                  cnt_v, idx_v, permb, rows_v, sem):
    c = lax.axis_index("c")
    s = lax.axis_index("s")
    w = s * NC + c

    zeros16 = jnp.zeros((16,), jnp.float32)

    def zbody(i, _):
        cnt_v[pl.ds(pl.multiple_of(i * 16, 16), 16)] = zeros16
        return 0
    lax.fori_loop(0, NPAD // 16, zbody, 0)

    pltpu.sync_copy(dst_hbm.at[pl.ds(pl.multiple_of(w * EPW, 8), EPW)], idx_v)

    ones16 = jnp.ones((16,), jnp.float32)

    def cbody(i, _):
        idx = idx_v[pl.ds(pl.multiple_of(i * 16, 16), 16)]
        plsc.addupdate_scatter(cnt_v, [idx], ones16)
        return 0
    lax.fori_loop(0, EPW // 16, cbody, 0)

    pltpu.sync_copy(cnt_v, degp_hbm.at[w])

    for k in range(RPW // CHA):
        base = w * RPW + k * CHA
        pltpu.sync_copy(perm_hbm.at[pl.ds(pl.multiple_of(base, 8), CHA)], permb)
        pltpu.async_copy(x_hbm.at[permb], rows_v, sem).wait()
        pltpu.sync_copy(rows_v, xn_hbm.at[pl.ds(pl.multiple_of(base, 8), CHA)])


def _stage_a(dstp, permp, xp):
    f = pl.kernel(
        _stage_a_body,
        out_type=(jax.ShapeDtypeStruct((NW, NPAD), jnp.float32),
                  jax.ShapeDtypeStruct((NPAD, D), jnp.float32)),
        mesh=_mesh(),
        scratch_types=[
            pltpu.VMEM((NPAD,), jnp.float32),
            pltpu.VMEM((EPW,), jnp.int32),
            pltpu.VMEM((CHA,), jnp.int32),
            pltpu.VMEM((CHA, D), jnp.float32),
            pltpu.SemaphoreType.DMA,
        ],
        compiler_params=pltpu.CompilerParams(needs_layout_passes=False),
    )
    return f(dstp, permp, xp)


def _scatter_body(src_hbm, dst2_hbm, g_hbm, zrow_hbm, out_hbm,
                  srcall, dstall, rows0, rows1, acc_sh, sg0, sg1):
    c = lax.axis_index("c")
    s = lax.axis_index("s")
    goff = c * NPAD

    pltpu.sync_copy(zrow_hbm, rows0)
    for k in range(RPT // CH):
        pltpu.sync_copy(
            rows0, acc_sh.at[pl.ds(pl.multiple_of(s * RPT + k * CH, 8), CH)])
    plsc.subcore_barrier()

    def fire_g(j, rows, sem):
        pltpu.async_copy(
            g_hbm.at[srcall.at[pl.ds(pl.multiple_of(j * CH, 8), CH)]],
            rows, sem)

    def wait_g(j, rows, sem):
        pltpu.make_async_copy(
            g_hbm.at[srcall.at[pl.ds(pl.multiple_of(j * CH, 8), CH)]],
            rows, sem).wait()

    def scat(j, rows):
        pltpu.sync_copy(rows, acc_sh.at[dstall.at[j]], add=True)

    for seg in range(NSEG):
        ebase = pl.multiple_of(s * EPT + seg * SEG, 8)
        rbase = s * (EPT // CH) + seg * SEGC
        pltpu.sync_copy(src_hbm.at[pl.ds(ebase, SEG)], srcall)
        pltpu.sync_copy(dst2_hbm.at[pl.ds(rbase, SEGC)], dstall)

        def addoff(i, _):
            o = pl.multiple_of(i * 16, 16)
            srcall[pl.ds(o, 16)] = srcall[pl.ds(o, 16)] + goff
            return 0
        lax.fori_loop(0, SEG // 16, addoff, 0)

        fire_g(0, rows0, sg0)

        def pair(jj, _):
            j0 = jj * 2
            fire_g(j0 + 1, rows1, sg1)
            wait_g(j0, rows0, sg0)
            scat(j0, rows0)

            @pl.when(jj < SEGP - 1)
            def _():
                fire_g(j0 + 2, rows0, sg0)

            wait_g(j0 + 1, rows1, sg1)
            scat(j0 + 1, rows1)
            return 0
        lax.fori_loop(0, SEGP, pair, 0)

    plsc.subcore_barrier()
    for k in range(RPT // CH):
        rr = pl.multiple_of(s * RPT + k * CH, 8)
        pltpu.sync_copy(acc_sh.at[pl.ds(rr, CH)], rows0)
        pltpu.sync_copy(rows0, out_hbm.at[pl.ds(goff + rr, CH)])


def _scatter(srcp, dst2d, gflat, zrow):
    f = pl.kernel(
        _scatter_body,
        out_type=jax.ShapeDtypeStruct((NC * NPAD, D), jnp.float32),
        mesh=_mesh(),
        scratch_types=[
            pltpu.VMEM((SEG,), jnp.int32),
            pltpu.VMEM((SEGC, CH), jnp.int32),
            pltpu.VMEM((CH, D), jnp.float32),
            pltpu.VMEM((CH, D), jnp.float32),
            pltpu.VMEM_SHARED((NPAD, D), jnp.float32),
            pltpu.SemaphoreType.DMA,
            pltpu.SemaphoreType.DMA,
        ],
        compiler_params=pltpu.CompilerParams(needs_layout_passes=False),
    )
    return f(srcp, dst2d, gflat, zrow)


def _prep1_body(degp_ref, xp_ref, xn_ref, w1_ref, g_ref, dinv_ref):
    deg = jnp.sum(degp_ref[...], axis=0) + 1.0
    dv = lax.rsqrt(deg)[:, None]
    dinv_ref[...] = dv
    g_ref[0] = dv * jnp.dot(xp_ref[...], w1_ref[...],
                            preferred_element_type=jnp.float32)
    g_ref[1] = dv * jnp.dot(xn_ref[...], w1_ref[...],
                            preferred_element_type=jnp.float32)


def _prep1(degp, xp, xn, W1):
    return pl.pallas_call(
        _prep1_body,
        grid=(GRID,),
        in_specs=[pl.BlockSpec((NW, BLK), lambda i: (0, i)),
                  pl.BlockSpec((BLK, D), lambda i: (i, 0)),
                  pl.BlockSpec((BLK, D), lambda i: (i, 0)),
                  pl.BlockSpec((D, D), lambda i: (0, 0))],
        out_specs=[pl.BlockSpec((2, BLK, D), lambda i: (0, i, 0)),
                   pl.BlockSpec((BLK, 1), lambda i: (i, 0))],
        out_shape=[jax.ShapeDtypeStruct((2, NPAD, D), jnp.float32),
                   jax.ShapeDtypeStruct((NPAD, 1), jnp.float32)],
    )(degp, xp, xn, W1)


def _prep2_body(acc_ref, g_ref, dinv_ref, w2_ref, b_ref, a_ref, o_ref):
    dv = dinv_ref[...]
    for k in range(2):
        z = dv * (acc_ref[k] + g_ref[k]) + b_ref[...]
        z = jnp.where(z >= 0, z, a_ref[...] * z)
        o_ref[k] = dv * jnp.dot(z, w2_ref[...],
                                preferred_element_type=jnp.float32)


def _prep2(acc, g, dinv, W2, b1, a1):
    return pl.pallas_call(
        _prep2_body,
        grid=(GRID,),
        in_specs=[pl.BlockSpec((2, BLK, D), lambda i: (0, i, 0)),
                  pl.BlockSpec((2, BLK, D), lambda i: (0, i, 0)),
                  pl.BlockSpec((BLK, 1), lambda i: (i, 0)),
                  pl.BlockSpec((D, D), lambda i: (0, 0)),
                  pl.BlockSpec((1, D), lambda i: (0, 0)),
                  pl.BlockSpec((1, D), lambda i: (0, 0))],
        out_specs=pl.BlockSpec((2, BLK, D), lambda i: (0, i, 0)),
        out_shape=jax.ShapeDtypeStruct((2, NPAD, D), jnp.float32),
    )(acc, g, dinv, W2, b1, a1)


def _final_body(acc_ref, g_ref, dinv_ref, b_ref, a_ref,
                pos_ref, neg_ref, sum_ref):
    i = pl.program_id(0)
    dv = dinv_ref[...]
    zp = dv * (acc_ref[0] + g_ref[0]) + b_ref[...]
    zp = jnp.where(zp >= 0, zp, a_ref[...] * zp)
    zn = dv * (acc_ref[1] + g_ref[1]) + b_ref[...]
    zn = jnp.where(zn >= 0, zn, a_ref[...] * zn)
    pos_ref[...] = zp
    neg_ref[...] = zn
    rows = lax.broadcasted_iota(jnp.int32, (BLK, 1), 0) + i * BLK
    part = jnp.sum(jnp.where(rows < N, zp, 0.0), axis=0, keepdims=True)

    @pl.when(i == 0)
    def _():
        sum_ref[...] = jnp.zeros_like(sum_ref)
    sum_ref[...] += part

    @pl.when(i == GRID - 1)
    def _():
        sum_ref[...] = jax.nn.sigmoid(sum_ref[...] * (1.0 / N))


def _final(acc, g, dinv, b2, a2):
    return pl.pallas_call(
        _final_body,
        grid=(GRID,),
        in_specs=[pl.BlockSpec((2, BLK, D), lambda i: (0, i, 0)),
                  pl.BlockSpec((2, BLK, D), lambda i: (0, i, 0)),
                  pl.BlockSpec((BLK, 1), lambda i: (i, 0)),
                  pl.BlockSpec((1, D), lambda i: (0, 0)),
                  pl.BlockSpec((1, D), lambda i: (0, 0))],
        out_specs=[pl.BlockSpec((BLK, D), lambda i: (i, 0)),
                   pl.BlockSpec((BLK, D), lambda i: (i, 0)),
                   pl.BlockSpec((1, D), lambda i: (0, 0))],
        out_shape=[jax.ShapeDtypeStruct((NPAD, D), jnp.float32),
                   jax.ShapeDtypeStruct((NPAD, D), jnp.float32),
                   jax.ShapeDtypeStruct((1, D), jnp.float32)],
    )(acc, g, dinv, b2, a2)


def kernel(x, edge_index, perm, W1, b1, a1, W2, b2, a2):
    x = x.astype(jnp.float32)
    srcp = jnp.pad(edge_index[0].astype(jnp.int32), (0, EPAD - E))
    dstp = jnp.pad(edge_index[1].astype(jnp.int32), (0, EPAD - E),
                   constant_values=N)
    dst2d = dstp.reshape(EPAD // CH, CH)
    permp = jnp.pad(perm.astype(jnp.int32), (0, NPAD - N))
    xp = jnp.pad(x, ((0, NPAD - N), (0, 0)))
    zrow = jnp.zeros((CH, D), jnp.float32)
    b1r = b1.reshape(1, D)
    a1r = a1.reshape(1, D)
    b2r = b2.reshape(1, D)
    a2r = a2.reshape(1, D)

    degp, xn = _stage_a(dstp, permp, xp)
    g1, dinv = _prep1(degp, xp, xn, W1)
    acc1 = _scatter(srcp, dst2d, g1.reshape(NC * NPAD, D), zrow)
    g2 = _prep2(acc1.reshape(2, NPAD, D), g1, dinv, W2, b1r, a1r)
    acc2 = _scatter(srcp, dst2d, g2.reshape(NC * NPAD, D), zrow)
    posz, negz, summ = _final(acc2.reshape(2, NPAD, D), g2, dinv, b2r, a2r)
    return posz[:N], negz[:N], summ.reshape(D)


# speedup vs baseline: 1.9626x; 1.9626x over previous
import jax
import jax.numpy as jnp
from jax import lax
from jax.experimental import pallas as pl
from jax.experimental.pallas import tpu as pltpu
from jax.experimental.pallas import tpu_sc as plsc

N = 10000
E = 320000
D = 128
NPAD = 10240
NC = 2
NS = 16
NW = NC * NS

BLK = 512
GRID = NPAD // BLK

EPAD = E
EPW = EPAD // NW
EPT = EPAD // NS
CH = 80
SEG = 4000
NSEG = EPT // SEG
SEGC = SEG // CH
SEGP = SEGC // 2
RPT = NPAD // NS
RPW = NPAD // NW
CHA = 64


def _mesh():
    return plsc.VectorSubcoreMesh(
        core_axis_name="c", subcore_axis_name="s",
        num_cores=NC, num_subcores=NS)


def _stage_a_body(dst_hbm, perm_hbm, x_hbm, degp_hbm, xn_hbm,
                  cnt_v, idx_v, permb, rows_v, sem):
    c = lax.axis_index("c")
    s = lax.axis_index("s")
    w = s * NC + c

    zeros16 = jnp.zeros((16,), jnp.float32)

    def zbody(i, _):
        cnt_v[pl.ds(pl.multiple_of(i * 16, 16), 16)] = zeros16
        return 0
    lax.fori_loop(0, NPAD // 16, zbody, 0)

    pltpu.sync_copy(dst_hbm.at[pl.ds(pl.multiple_of(w * EPW, 8), EPW)], idx_v)

    ones16 = jnp.ones((16,), jnp.float32)

    def cbody(i, _):
        idx = idx_v[pl.ds(pl.multiple_of(i * 16, 16), 16)]
        plsc.addupdate_scatter(cnt_v, [idx], ones16)
        return 0
    lax.fori_loop(0, EPW // 16, cbody, 0)

    pltpu.sync_copy(cnt_v, degp_hbm.at[w])

    for k in range(RPW // CHA):
        base = w * RPW + k * CHA
        pltpu.sync_copy(perm_hbm.at[pl.ds(pl.multiple_of(base, 8), CHA)], permb)
        pltpu.async_copy(x_hbm.at[permb], rows_v, sem).wait()
        pltpu.sync_copy(rows_v, xn_hbm.at[pl.ds(pl.multiple_of(base, 8), CHA)])


def _stage_a(dstp, permp, xp):
    f = pl.kernel(
        _stage_a_body,
        out_type=(jax.ShapeDtypeStruct((NW, NPAD), jnp.float32),
                  jax.ShapeDtypeStruct((NPAD, D), jnp.float32)),
        mesh=_mesh(),
        scratch_types=[
            pltpu.VMEM((NPAD,), jnp.float32),
            pltpu.VMEM((EPW,), jnp.int32),
            pltpu.VMEM((CHA,), jnp.int32),
            pltpu.VMEM((CHA, D), jnp.float32),
            pltpu.SemaphoreType.DMA,
        ],
        compiler_params=pltpu.CompilerParams(needs_layout_passes=False),
    )
    return f(dstp, permp, xp)


def _scatter_body(src_hbm, dst_hbm, g_hbm, zrow_hbm, out_hbm,
                  srcall, dstall, dstb0, dstb1,
                  rows0, rows1, acc_sh, sg0, sg1, sd0, sd1):
    c = lax.axis_index("c")
    s = lax.axis_index("s")
    goff = c * NPAD

    pltpu.sync_copy(zrow_hbm, rows0)
    for k in range(RPT // CH):
        pltpu.async_copy(
            rows0, acc_sh.at[pl.ds(pl.multiple_of(s * RPT + k * CH, 8), CH)],
            sd0)
    for k in range(RPT // CH):
        pltpu.make_async_copy(
            rows0, acc_sh.at[pl.ds(pl.multiple_of(s * RPT + k * CH, 8), CH)],
            sd0).wait()
    plsc.subcore_barrier()

    def stage_dst(j, db):
        for t in range(CH // 16):
            o = pl.multiple_of(j * CH + t * 16, 16)
            db[pl.ds(t * 16, 16)] = dstall[pl.ds(o, 16)]

    def fire_g(j, rows, sem):
        pltpu.async_copy(
            g_hbm.at[srcall.at[pl.ds(pl.multiple_of(j * CH, 8), CH)]],
            rows, sem)

    def wait_g(j, rows, sem):
        pltpu.make_async_copy(
            g_hbm.at[srcall.at[pl.ds(pl.multiple_of(j * CH, 8), CH)]],
            rows, sem).wait()

    def scat(db, rows):
        pltpu.sync_copy(rows, acc_sh.at[db], add=True)

    for seg in range(NSEG):
        ebase = pl.multiple_of(s * EPT + seg * SEG, 8)
        pltpu.sync_copy(src_hbm.at[pl.ds(ebase, SEG)], srcall)
        pltpu.sync_copy(dst_hbm.at[pl.ds(ebase, SEG)], dstall)

        def addoff(i, _):
            o = pl.multiple_of(i * 16, 16)
            srcall[pl.ds(o, 16)] = srcall[pl.ds(o, 16)] + goff
            return 0
        lax.fori_loop(0, SEG // 16, addoff, 0)

        stage_dst(0, dstb0)
        fire_g(0, rows0, sg0)

        def pair(jj, _):
            j0 = jj * 2
            stage_dst(j0 + 1, dstb1)
            fire_g(j0 + 1, rows1, sg1)
            wait_g(j0, rows0, sg0)
            scat(dstb0, rows0)

            @pl.when(jj < SEGP - 1)
            def _():
                stage_dst(j0 + 2, dstb0)
                fire_g(j0 + 2, rows0, sg0)

            wait_g(j0 + 1, rows1, sg1)
            scat(dstb1, rows1)
            return 0
        lax.fori_loop(0, SEGP, pair, 0)

    plsc.subcore_barrier()
    drn = RPT // CH
    for k in range(drn):
        rr = pl.multiple_of(s * RPT + k * CH, 8)
        rows, sem = (rows0, sd0) if k % 2 == 0 else (rows1, sd1)
        pltpu.sync_copy(acc_sh.at[pl.ds(rr, CH)], rows)
        pltpu.async_copy(rows, out_hbm.at[pl.ds(goff + rr, CH)], sem)
        if k >= 1:
            rp = pl.multiple_of(s * RPT + (k - 1) * CH, 8)
            rowsp, semp = (rows0, sd0) if (k - 1) % 2 == 0 else (rows1, sd1)
            pltpu.make_async_copy(
                rowsp, out_hbm.at[pl.ds(goff + rp, CH)], semp).wait()
    rl = pl.multiple_of(s * RPT + (drn - 1) * CH, 8)
    rowsl, seml = (rows0, sd0) if (drn - 1) % 2 == 0 else (rows1, sd1)
    pltpu.make_async_copy(
        rowsl, out_hbm.at[pl.ds(goff + rl, CH)], seml).wait()


def _scatter(srcp, dstp, gflat, zrow):
    f = pl.kernel(
        _scatter_body,
        out_type=jax.ShapeDtypeStruct((NC * NPAD, D), jnp.float32),
        mesh=_mesh(),
        scratch_types=[
            pltpu.VMEM((SEG,), jnp.int32),
            pltpu.VMEM((SEG,), jnp.int32),
            pltpu.VMEM((CH,), jnp.int32),
            pltpu.VMEM((CH,), jnp.int32),
            pltpu.VMEM((CH, D), jnp.float32),
            pltpu.VMEM((CH, D), jnp.float32),
            pltpu.VMEM_SHARED((NPAD, D), jnp.float32),
            pltpu.SemaphoreType.DMA,
            pltpu.SemaphoreType.DMA,
            pltpu.SemaphoreType.DMA,
            pltpu.SemaphoreType.DMA,
        ],
        compiler_params=pltpu.CompilerParams(needs_layout_passes=False),
    )
    return f(srcp, dstp, gflat, zrow)


def _prep1_body(degp_ref, xp_ref, xn_ref, w1_ref, g_ref, dinv_ref):
    deg = jnp.sum(degp_ref[...], axis=0) + 1.0
    dv = lax.rsqrt(deg)[:, None]
    dinv_ref[...] = dv
    g_ref[0] = dv * jnp.dot(xp_ref[...], w1_ref[...],
                            preferred_element_type=jnp.float32)
    g_ref[1] = dv * jnp.dot(xn_ref[...], w1_ref[...],
                            preferred_element_type=jnp.float32)


def _prep1(degp, xp, xn, W1):
    return pl.pallas_call(
        _prep1_body,
        grid=(GRID,),
        in_specs=[pl.BlockSpec((NW, BLK), lambda i: (0, i)),
                  pl.BlockSpec((BLK, D), lambda i: (i, 0)),
                  pl.BlockSpec((BLK, D), lambda i: (i, 0)),
                  pl.BlockSpec((D, D), lambda i: (0, 0))],
        out_specs=[pl.BlockSpec((2, BLK, D), lambda i: (0, i, 0)),
                   pl.BlockSpec((BLK, 1), lambda i: (i, 0))],
        out_shape=[jax.ShapeDtypeStruct((2, NPAD, D), jnp.float32),
                   jax.ShapeDtypeStruct((NPAD, 1), jnp.float32)],
    )(degp, xp, xn, W1)


def _prep2_body(acc_ref, g_ref, dinv_ref, w2_ref, b_ref, a_ref, o_ref):
    dv = dinv_ref[...]
    for k in range(2):
        z = dv * (acc_ref[k] + g_ref[k]) + b_ref[...]
        z = jnp.where(z >= 0, z, a_ref[...] * z)
        o_ref[k] = dv * jnp.dot(z, w2_ref[...],
                                preferred_element_type=jnp.float32)


def _prep2(acc, g, dinv, W2, b1, a1):
    return pl.pallas_call(
        _prep2_body,
        grid=(GRID,),
        in_specs=[pl.BlockSpec((2, BLK, D), lambda i: (0, i, 0)),
                  pl.BlockSpec((2, BLK, D), lambda i: (0, i, 0)),
                  pl.BlockSpec((BLK, 1), lambda i: (i, 0)),
                  pl.BlockSpec((D, D), lambda i: (0, 0)),
                  pl.BlockSpec((1, D), lambda i: (0, 0)),
                  pl.BlockSpec((1, D), lambda i: (0, 0))],
        out_specs=pl.BlockSpec((2, BLK, D), lambda i: (0, i, 0)),
        out_shape=jax.ShapeDtypeStruct((2, NPAD, D), jnp.float32),
    )(acc, g, dinv, W2, b1, a1)


def _final_body(acc_ref, g_ref, dinv_ref, b_ref, a_ref,
                pos_ref, neg_ref, sum_ref):
    i = pl.program_id(0)
    dv = dinv_ref[...]
    zp = dv * (acc_ref[0] + g_ref[0]) + b_ref[...]
    zp = jnp.where(zp >= 0, zp, a_ref[...] * zp)
    zn = dv * (acc_ref[1] + g_ref[1]) + b_ref[...]
    zn = jnp.where(zn >= 0, zn, a_ref[...] * zn)
    pos_ref[...] = zp
    neg_ref[...] = zn
    rows = lax.broadcasted_iota(jnp.int32, (BLK, 1), 0) + i * BLK
    part = jnp.sum(jnp.where(rows < N, zp, 0.0), axis=0, keepdims=True)

    @pl.when(i == 0)
    def _():
        sum_ref[...] = jnp.zeros_like(sum_ref)
    sum_ref[...] += part

    @pl.when(i == GRID - 1)
    def _():
        sum_ref[...] = jax.nn.sigmoid(sum_ref[...] * (1.0 / N))


def _final(acc, g, dinv, b2, a2):
    return pl.pallas_call(
        _final_body,
        grid=(GRID,),
        in_specs=[pl.BlockSpec((2, BLK, D), lambda i: (0, i, 0)),
                  pl.BlockSpec((2, BLK, D), lambda i: (0, i, 0)),
                  pl.BlockSpec((BLK, 1), lambda i: (i, 0)),
                  pl.BlockSpec((1, D), lambda i: (0, 0)),
                  pl.BlockSpec((1, D), lambda i: (0, 0))],
        out_specs=[pl.BlockSpec((BLK, D), lambda i: (i, 0)),
                   pl.BlockSpec((BLK, D), lambda i: (i, 0)),
                   pl.BlockSpec((1, D), lambda i: (0, 0))],
        out_shape=[jax.ShapeDtypeStruct((NPAD, D), jnp.float32),
                   jax.ShapeDtypeStruct((NPAD, D), jnp.float32),
                   jax.ShapeDtypeStruct((1, D), jnp.float32)],
    )(acc, g, dinv, b2, a2)


def kernel(x, edge_index, perm, W1, b1, a1, W2, b2, a2):
    x = x.astype(jnp.float32)
    srcp = edge_index[0].astype(jnp.int32)
    dstp = edge_index[1].astype(jnp.int32)
    permp = jnp.pad(perm.astype(jnp.int32), (0, NPAD - N))
    xp = jnp.pad(x, ((0, NPAD - N), (0, 0)))
    zrow = jnp.zeros((CH, D), jnp.float32)
    b1r = b1.reshape(1, D)
    a1r = a1.reshape(1, D)
    b2r = b2.reshape(1, D)
    a2r = a2.reshape(1, D)

    degp, xn = _stage_a(dstp, permp, xp)
    g1, dinv = _prep1(degp, xp, xn, W1)
    acc1 = _scatter(srcp, dstp, g1.reshape(NC * NPAD, D), zrow)
    g2 = _prep2(acc1.reshape(2, NPAD, D), g1, dinv, W2, b1r, a1r)
    acc2 = _scatter(srcp, dstp, g2.reshape(NC * NPAD, D), zrow)
    posz, negz, summ = _final(acc2.reshape(2, NPAD, D), g2, dinv, b2r, a2r)
    return posz[:N], negz[:N], summ.reshape(D)


# speedup vs baseline: 2.0158x; 1.0271x over previous
import jax
import jax.numpy as jnp
from jax import lax
from jax.experimental import pallas as pl
from jax.experimental.pallas import tpu as pltpu
from jax.experimental.pallas import tpu_sc as plsc

N = 10000
E = 320000
D = 128
NPAD = 10240
NC = 2
NS = 16
NW = NC * NS

BLK = 1024
GRID = NPAD // BLK

EPAD = E
EPW = EPAD // NW
EPT = EPAD // NS
CH = 80
SEG = 4000
NSEG = EPT // SEG
SEGC = SEG // CH
SEGP = SEGC // 2
RPT = NPAD // NS
RPW = NPAD // NW
CHA = 64


def _mesh():
    return plsc.VectorSubcoreMesh(
        core_axis_name="c", subcore_axis_name="s",
        num_cores=NC, num_subcores=NS)


def _stage_a_body(dst_hbm, perm_hbm, x_hbm, degp_hbm, xn_hbm,
                  cnt_v, idx_v, permb, rows_v, sem):
    c = lax.axis_index("c")
    s = lax.axis_index("s")
    w = s * NC + c

    zeros16 = jnp.zeros((16,), jnp.float32)

    def zbody(i, _):
        cnt_v[pl.ds(pl.multiple_of(i * 16, 16), 16)] = zeros16
        return 0
    lax.fori_loop(0, NPAD // 16, zbody, 0)

    pltpu.sync_copy(dst_hbm.at[pl.ds(pl.multiple_of(w * EPW, 8), EPW)], idx_v)

    ones16 = jnp.ones((16,), jnp.float32)

    def cbody(i, _):
        idx = idx_v[pl.ds(pl.multiple_of(i * 16, 16), 16)]
        plsc.addupdate_scatter(cnt_v, [idx], ones16)
        return 0
    lax.fori_loop(0, EPW // 16, cbody, 0)

    pltpu.sync_copy(cnt_v, degp_hbm.at[w])

    for k in range(RPW // CHA):
        base = w * RPW + k * CHA
        pltpu.sync_copy(perm_hbm.at[pl.ds(pl.multiple_of(base, 8), CHA)], permb)
        pltpu.async_copy(x_hbm.at[permb], rows_v, sem).wait()
        pltpu.sync_copy(rows_v, xn_hbm.at[pl.ds(pl.multiple_of(base, 8), CHA)])


def _stage_a(dstp, permp, xp):
    f = pl.kernel(
        _stage_a_body,
        out_type=(jax.ShapeDtypeStruct((NW, NPAD), jnp.float32),
                  jax.ShapeDtypeStruct((NPAD, D), jnp.float32)),
        mesh=_mesh(),
        scratch_types=[
            pltpu.VMEM((NPAD,), jnp.float32),
            pltpu.VMEM((EPW,), jnp.int32),
            pltpu.VMEM((CHA,), jnp.int32),
            pltpu.VMEM((CHA, D), jnp.float32),
            pltpu.SemaphoreType.DMA,
        ],
        compiler_params=pltpu.CompilerParams(needs_layout_passes=False),
    )
    return f(dstp, permp, xp)


def _scatter_body(src_hbm, dst_hbm, g_hbm, zrow_hbm, out_hbm,
                  srcall, dstall, dstb0, dstb1,
                  rows0, rows1, acc_sh, sg0, sg1, sd0, sd1):
    c = lax.axis_index("c")
    s = lax.axis_index("s")
    goff = c * NPAD

    pltpu.sync_copy(zrow_hbm, rows0)
    for k in range(RPT // CH):
        pltpu.async_copy(
            rows0, acc_sh.at[pl.ds(pl.multiple_of(s * RPT + k * CH, 8), CH)],
            sd0)
    for k in range(RPT // CH):
        pltpu.make_async_copy(
            rows0, acc_sh.at[pl.ds(pl.multiple_of(s * RPT + k * CH, 8), CH)],
            sd0).wait()
    plsc.subcore_barrier()

    def stage_dst(j, db):
        for t in range(CH // 16):
            o = pl.multiple_of(j * CH + t * 16, 16)
            db[pl.ds(t * 16, 16)] = dstall[pl.ds(o, 16)]

    def fire_g(j, rows, sem):
        pltpu.async_copy(
            g_hbm.at[srcall.at[pl.ds(pl.multiple_of(j * CH, 8), CH)]],
            rows, sem)

    def wait_g(j, rows, sem):
        pltpu.make_async_copy(
            g_hbm.at[srcall.at[pl.ds(pl.multiple_of(j * CH, 8), CH)]],
            rows, sem).wait()

    def scat(db, rows):
        pltpu.sync_copy(rows, acc_sh.at[db], add=True)

    for seg in range(NSEG):
        ebase = pl.multiple_of(s * EPT + seg * SEG, 8)
        pltpu.sync_copy(src_hbm.at[pl.ds(ebase, SEG)], srcall)
        pltpu.sync_copy(dst_hbm.at[pl.ds(ebase, SEG)], dstall)

        def addoff(i, _):
            o = pl.multiple_of(i * 16, 16)
            srcall[pl.ds(o, 16)] = srcall[pl.ds(o, 16)] + goff
            return 0
        lax.fori_loop(0, SEG // 16, addoff, 0)

        stage_dst(0, dstb0)
        fire_g(0, rows0, sg0)

        def pair(jj, _):
            j0 = jj * 2
            stage_dst(j0 + 1, dstb1)
            fire_g(j0 + 1, rows1, sg1)
            wait_g(j0, rows0, sg0)
            scat(dstb0, rows0)

            @pl.when(jj < SEGP - 1)
            def _():
                stage_dst(j0 + 2, dstb0)
                fire_g(j0 + 2, rows0, sg0)

            wait_g(j0 + 1, rows1, sg1)
            scat(dstb1, rows1)
            return 0
        lax.fori_loop(0, SEGP, pair, 0)

    plsc.subcore_barrier()
    drn = RPT // CH
    for k in range(drn):
        rr = pl.multiple_of(s * RPT + k * CH, 8)
        rows, sem = (rows0, sd0) if k % 2 == 0 else (rows1, sd1)
        pltpu.sync_copy(acc_sh.at[pl.ds(rr, CH)], rows)
        pltpu.async_copy(rows, out_hbm.at[pl.ds(goff + rr, CH)], sem)
        if k >= 1:
            rp = pl.multiple_of(s * RPT + (k - 1) * CH, 8)
            rowsp, semp = (rows0, sd0) if (k - 1) % 2 == 0 else (rows1, sd1)
            pltpu.make_async_copy(
                rowsp, out_hbm.at[pl.ds(goff + rp, CH)], semp).wait()
    rl = pl.multiple_of(s * RPT + (drn - 1) * CH, 8)
    rowsl, seml = (rows0, sd0) if (drn - 1) % 2 == 0 else (rows1, sd1)
    pltpu.make_async_copy(
        rowsl, out_hbm.at[pl.ds(goff + rl, CH)], seml).wait()


def _scatter(srcp, dstp, gflat, zrow):
    f = pl.kernel(
        _scatter_body,
        out_type=jax.ShapeDtypeStruct((NC * NPAD, D), jnp.float32),
        mesh=_mesh(),
        scratch_types=[
            pltpu.VMEM((SEG,), jnp.int32),
            pltpu.VMEM((SEG,), jnp.int32),
            pltpu.VMEM((CH,), jnp.int32),
            pltpu.VMEM((CH,), jnp.int32),
            pltpu.VMEM((CH, D), jnp.float32),
            pltpu.VMEM((CH, D), jnp.float32),
            pltpu.VMEM_SHARED((NPAD, D), jnp.float32),
            pltpu.SemaphoreType.DMA,
            pltpu.SemaphoreType.DMA,
            pltpu.SemaphoreType.DMA,
            pltpu.SemaphoreType.DMA,
        ],
        compiler_params=pltpu.CompilerParams(needs_layout_passes=False),
    )
    return f(srcp, dstp, gflat, zrow)


def _prep1_body(degp_ref, xp_ref, xn_ref, w1_ref, g_ref, dinv_ref):
    deg = jnp.sum(degp_ref[...], axis=0) + 1.0
    dv = lax.rsqrt(deg)[:, None]
    dinv_ref[...] = dv
    g_ref[0] = dv * jnp.dot(xp_ref[...], w1_ref[...],
                            preferred_element_type=jnp.float32)
    g_ref[1] = dv * jnp.dot(xn_ref[...], w1_ref[...],
                            preferred_element_type=jnp.float32)


def _prep1(degp, xp, xn, W1):
    return pl.pallas_call(
        _prep1_body,
        grid=(GRID,),
        in_specs=[pl.BlockSpec((NW, BLK), lambda i: (0, i)),
                  pl.BlockSpec((BLK, D), lambda i: (i, 0)),
                  pl.BlockSpec((BLK, D), lambda i: (i, 0)),
                  pl.BlockSpec((D, D), lambda i: (0, 0))],
        out_specs=[pl.BlockSpec((2, BLK, D), lambda i: (0, i, 0)),
                   pl.BlockSpec((BLK, 1), lambda i: (i, 0))],
        out_shape=[jax.ShapeDtypeStruct((2, NPAD, D), jnp.float32),
                   jax.ShapeDtypeStruct((NPAD, 1), jnp.float32)],
    )(degp, xp, xn, W1)


def _prep2_body(acc_ref, g_ref, dinv_ref, w2_ref, b_ref, a_ref, o_ref):
    dv = dinv_ref[...]
    for k in range(2):
        z = dv * (acc_ref[k] + g_ref[k]) + b_ref[...]
        z = jnp.where(z >= 0, z, a_ref[...] * z)
        o_ref[k] = dv * jnp.dot(z, w2_ref[...],
                                preferred_element_type=jnp.float32)


def _prep2(acc, g, dinv, W2, b1, a1):
    return pl.pallas_call(
        _prep2_body,
        grid=(GRID,),
        in_specs=[pl.BlockSpec((2, BLK, D), lambda i: (0, i, 0)),
                  pl.BlockSpec((2, BLK, D), lambda i: (0, i, 0)),
                  pl.BlockSpec((BLK, 1), lambda i: (i, 0)),
                  pl.BlockSpec((D, D), lambda i: (0, 0)),
                  pl.BlockSpec((1, D), lambda i: (0, 0)),
                  pl.BlockSpec((1, D), lambda i: (0, 0))],
        out_specs=pl.BlockSpec((2, BLK, D), lambda i: (0, i, 0)),
        out_shape=jax.ShapeDtypeStruct((2, NPAD, D), jnp.float32),
    )(acc, g, dinv, W2, b1, a1)


def _final_body(acc_ref, g_ref, dinv_ref, b_ref, a_ref,
                pos_ref, neg_ref, sum_ref):
    i = pl.program_id(0)
    dv = dinv_ref[...]
    zp = dv * (acc_ref[0] + g_ref[0]) + b_ref[...]
    zp = jnp.where(zp >= 0, zp, a_ref[...] * zp)
    zn = dv * (acc_ref[1] + g_ref[1]) + b_ref[...]
    zn = jnp.where(zn >= 0, zn, a_ref[...] * zn)
    pos_ref[...] = zp
    neg_ref[...] = zn
    rows = lax.broadcasted_iota(jnp.int32, (BLK, 1), 0) + i * BLK
    part = jnp.sum(jnp.where(rows < N, zp, 0.0), axis=0, keepdims=True)

    @pl.when(i == 0)
    def _():
        sum_ref[...] = jnp.zeros_like(sum_ref)
    sum_ref[...] += part

    @pl.when(i == GRID - 1)
    def _():
        sum_ref[...] = jax.nn.sigmoid(sum_ref[...] * (1.0 / N))


def _final(acc, g, dinv, b2, a2):
    return pl.pallas_call(
        _final_body,
        grid=(GRID,),
        in_specs=[pl.BlockSpec((2, BLK, D), lambda i: (0, i, 0)),
                  pl.BlockSpec((2, BLK, D), lambda i: (0, i, 0)),
                  pl.BlockSpec((BLK, 1), lambda i: (i, 0)),
                  pl.BlockSpec((1, D), lambda i: (0, 0)),
                  pl.BlockSpec((1, D), lambda i: (0, 0))],
        out_specs=[pl.BlockSpec((BLK, D), lambda i: (i, 0)),
                   pl.BlockSpec((BLK, D), lambda i: (i, 0)),
                   pl.BlockSpec((1, D), lambda i: (0, 0))],
        out_shape=[jax.ShapeDtypeStruct((NPAD, D), jnp.float32),
                   jax.ShapeDtypeStruct((NPAD, D), jnp.float32),
                   jax.ShapeDtypeStruct((1, D), jnp.float32)],
    )(acc, g, dinv, b2, a2)


def kernel(x, edge_index, perm, W1, b1, a1, W2, b2, a2):
    x = x.astype(jnp.float32)
    srcp = edge_index[0].astype(jnp.int32)
    dstp = edge_index[1].astype(jnp.int32)
    permp = jnp.pad(perm.astype(jnp.int32), (0, NPAD - N))
    xp = jnp.pad(x, ((0, NPAD - N), (0, 0)))
    zrow = jnp.zeros((CH, D), jnp.float32)
    b1r = b1.reshape(1, D)
    a1r = a1.reshape(1, D)
    b2r = b2.reshape(1, D)
    a2r = a2.reshape(1, D)

    degp, xn = _stage_a(dstp, permp, xp)
    g1, dinv = _prep1(degp, xp, xn, W1)
    acc1 = _scatter(srcp, dstp, g1.reshape(NC * NPAD, D), zrow)
    g2 = _prep2(acc1.reshape(2, NPAD, D), g1, dinv, W2, b1r, a1r)
    acc2 = _scatter(srcp, dstp, g2.reshape(NC * NPAD, D), zrow)
    posz, negz, summ = _final(acc2.reshape(2, NPAD, D), g2, dinv, b2r, a2r)
    return posz[:N], negz[:N], summ.reshape(D)


# speedup vs baseline: 2.0353x; 1.0097x over previous
import jax
import jax.numpy as jnp
from jax import lax
from jax.experimental import pallas as pl
from jax.experimental.pallas import tpu as pltpu
from jax.experimental.pallas import tpu_sc as plsc

N = 10000
E = 320000
D = 128
NPAD = 10240
NC = 2
NS = 16
NW = NC * NS

BLK = 2048
GRID = NPAD // BLK

EPAD = E
EPW = EPAD // NW
EPT = EPAD // NS
CH = 80
SEG = 4000
NSEG = EPT // SEG
SEGC = SEG // CH
SEGP = SEGC // 2
RPT = NPAD // NS
RPW = NPAD // NW
CHA = 64


def _mesh():
    return plsc.VectorSubcoreMesh(
        core_axis_name="c", subcore_axis_name="s",
        num_cores=NC, num_subcores=NS)


def _stage_a_body(dst_hbm, perm_hbm, x_hbm, degp_hbm, xn_hbm,
                  cnt_v, idx_v, permb, rows_v, sem):
    c = lax.axis_index("c")
    s = lax.axis_index("s")
    w = s * NC + c

    zeros16 = jnp.zeros((16,), jnp.float32)

    def zbody(i, _):
        cnt_v[pl.ds(pl.multiple_of(i * 16, 16), 16)] = zeros16
        return 0
    lax.fori_loop(0, NPAD // 16, zbody, 0)

    pltpu.sync_copy(dst_hbm.at[pl.ds(pl.multiple_of(w * EPW, 8), EPW)], idx_v)

    ones16 = jnp.ones((16,), jnp.float32)

    def cbody(i, _):
        idx = idx_v[pl.ds(pl.multiple_of(i * 16, 16), 16)]
        plsc.addupdate_scatter(cnt_v, [idx], ones16)
        return 0
    lax.fori_loop(0, EPW // 16, cbody, 0)

    pltpu.sync_copy(cnt_v, degp_hbm.at[w])

    for k in range(RPW // CHA):
        base = w * RPW + k * CHA
        pltpu.sync_copy(perm_hbm.at[pl.ds(pl.multiple_of(base, 8), CHA)], permb)
        pltpu.async_copy(x_hbm.at[permb], rows_v, sem).wait()
        pltpu.sync_copy(rows_v, xn_hbm.at[pl.ds(pl.multiple_of(base, 8), CHA)])


def _stage_a(dstp, permp, xp):
    f = pl.kernel(
        _stage_a_body,
        out_type=(jax.ShapeDtypeStruct((NW, NPAD), jnp.float32),
                  jax.ShapeDtypeStruct((NPAD, D), jnp.float32)),
        mesh=_mesh(),
        scratch_types=[
            pltpu.VMEM((NPAD,), jnp.float32),
            pltpu.VMEM((EPW,), jnp.int32),
            pltpu.VMEM((CHA,), jnp.int32),
            pltpu.VMEM((CHA, D), jnp.float32),
            pltpu.SemaphoreType.DMA,
        ],
        compiler_params=pltpu.CompilerParams(needs_layout_passes=False),
    )
    return f(dstp, permp, xp)


def _scatter_body(src_hbm, dst_hbm, g_hbm, zrow_hbm, out_hbm,
                  srcall, dstall, dstb0, dstb1,
                  rows0, rows1, acc_sh, sg0, sg1, sd0, sd1):
    c = lax.axis_index("c")
    s = lax.axis_index("s")
    goff = c * NPAD

    pltpu.sync_copy(zrow_hbm, rows0)
    for k in range(RPT // CH):
        pltpu.async_copy(
            rows0, acc_sh.at[pl.ds(pl.multiple_of(s * RPT + k * CH, 8), CH)],
            sd0)
    for k in range(RPT // CH):
        pltpu.make_async_copy(
            rows0, acc_sh.at[pl.ds(pl.multiple_of(s * RPT + k * CH, 8), CH)],
            sd0).wait()
    plsc.subcore_barrier()

    def stage_dst(j, db):
        for t in range(CH // 16):
            o = pl.multiple_of(j * CH + t * 16, 16)
            db[pl.ds(t * 16, 16)] = dstall[pl.ds(o, 16)]

    def fire_g(j, rows, sem):
        pltpu.async_copy(
            g_hbm.at[srcall.at[pl.ds(pl.multiple_of(j * CH, 8), CH)]],
            rows, sem)

    def wait_g(j, rows, sem):
        pltpu.make_async_copy(
            g_hbm.at[srcall.at[pl.ds(pl.multiple_of(j * CH, 8), CH)]],
            rows, sem).wait()

    def scat(db, rows):
        pltpu.sync_copy(rows, acc_sh.at[db], add=True)

    for seg in range(NSEG):
        ebase = pl.multiple_of(s * EPT + seg * SEG, 8)
        pltpu.sync_copy(src_hbm.at[pl.ds(ebase, SEG)], srcall)
        pltpu.sync_copy(dst_hbm.at[pl.ds(ebase, SEG)], dstall)

        def addoff(i, _):
            o = pl.multiple_of(i * 16, 16)
            srcall[pl.ds(o, 16)] = srcall[pl.ds(o, 16)] + goff
            return 0
        lax.fori_loop(0, SEG // 16, addoff, 0)

        stage_dst(0, dstb0)
        fire_g(0, rows0, sg0)

        def pair(jj, _):
            j0 = jj * 2
            stage_dst(j0 + 1, dstb1)
            fire_g(j0 + 1, rows1, sg1)
            wait_g(j0, rows0, sg0)
            scat(dstb0, rows0)

            @pl.when(jj < SEGP - 1)
            def _():
                stage_dst(j0 + 2, dstb0)
                fire_g(j0 + 2, rows0, sg0)

            wait_g(j0 + 1, rows1, sg1)
            scat(dstb1, rows1)
            return 0
        lax.fori_loop(0, SEGP, pair, 0)

    plsc.subcore_barrier()
    drn = RPT // CH
    for k in range(drn):
        rr = pl.multiple_of(s * RPT + k * CH, 8)
        rows, sem = (rows0, sd0) if k % 2 == 0 else (rows1, sd1)
        pltpu.sync_copy(acc_sh.at[pl.ds(rr, CH)], rows)
        pltpu.async_copy(rows, out_hbm.at[pl.ds(goff + rr, CH)], sem)
        if k >= 1:
            rp = pl.multiple_of(s * RPT + (k - 1) * CH, 8)
            rowsp, semp = (rows0, sd0) if (k - 1) % 2 == 0 else (rows1, sd1)
            pltpu.make_async_copy(
                rowsp, out_hbm.at[pl.ds(goff + rp, CH)], semp).wait()
    rl = pl.multiple_of(s * RPT + (drn - 1) * CH, 8)
    rowsl, seml = (rows0, sd0) if (drn - 1) % 2 == 0 else (rows1, sd1)
    pltpu.make_async_copy(
        rowsl, out_hbm.at[pl.ds(goff + rl, CH)], seml).wait()


def _scatter(srcp, dstp, gflat, zrow):
    f = pl.kernel(
        _scatter_body,
        out_type=jax.ShapeDtypeStruct((NC * NPAD, D), jnp.float32),
        mesh=_mesh(),
        scratch_types=[
            pltpu.VMEM((SEG,), jnp.int32),
            pltpu.VMEM((SEG,), jnp.int32),
            pltpu.VMEM((CH,), jnp.int32),
            pltpu.VMEM((CH,), jnp.int32),
            pltpu.VMEM((CH, D), jnp.float32),
            pltpu.VMEM((CH, D), jnp.float32),
            pltpu.VMEM_SHARED((NPAD, D), jnp.float32),
            pltpu.SemaphoreType.DMA,
            pltpu.SemaphoreType.DMA,
            pltpu.SemaphoreType.DMA,
            pltpu.SemaphoreType.DMA,
        ],
        compiler_params=pltpu.CompilerParams(needs_layout_passes=False),
    )
    return f(srcp, dstp, gflat, zrow)


def _prep1_body(degp_ref, xp_ref, xn_ref, w1_ref, g_ref, dinv_ref):
    deg = jnp.sum(degp_ref[...], axis=0) + 1.0
    dv = lax.rsqrt(deg)[:, None]
    dinv_ref[...] = dv
    g_ref[0] = dv * jnp.dot(xp_ref[...], w1_ref[...],
                            preferred_element_type=jnp.float32)
    g_ref[1] = dv * jnp.dot(xn_ref[...], w1_ref[...],
                            preferred_element_type=jnp.float32)


def _prep1(degp, xp, xn, W1):
    return pl.pallas_call(
        _prep1_body,
        grid=(GRID,),
        in_specs=[pl.BlockSpec((NW, BLK), lambda i: (0, i)),
                  pl.BlockSpec((BLK, D), lambda i: (i, 0)),
                  pl.BlockSpec((BLK, D), lambda i: (i, 0)),
                  pl.BlockSpec((D, D), lambda i: (0, 0))],
        out_specs=[pl.BlockSpec((2, BLK, D), lambda i: (0, i, 0)),
                   pl.BlockSpec((BLK, 1), lambda i: (i, 0))],
        out_shape=[jax.ShapeDtypeStruct((2, NPAD, D), jnp.float32),
                   jax.ShapeDtypeStruct((NPAD, 1), jnp.float32)],
    )(degp, xp, xn, W1)


def _prep2_body(acc_ref, g_ref, dinv_ref, w2_ref, b_ref, a_ref, o_ref):
    dv = dinv_ref[...]
    for k in range(2):
        z = dv * (acc_ref[k] + g_ref[k]) + b_ref[...]
        z = jnp.where(z >= 0, z, a_ref[...] * z)
        o_ref[k] = dv * jnp.dot(z, w2_ref[...],
                                preferred_element_type=jnp.float32)


def _prep2(acc, g, dinv, W2, b1, a1):
    return pl.pallas_call(
        _prep2_body,
        grid=(GRID,),
        in_specs=[pl.BlockSpec((2, BLK, D), lambda i: (0, i, 0)),
                  pl.BlockSpec((2, BLK, D), lambda i: (0, i, 0)),
                  pl.BlockSpec((BLK, 1), lambda i: (i, 0)),
                  pl.BlockSpec((D, D), lambda i: (0, 0)),
                  pl.BlockSpec((1, D), lambda i: (0, 0)),
                  pl.BlockSpec((1, D), lambda i: (0, 0))],
        out_specs=pl.BlockSpec((2, BLK, D), lambda i: (0, i, 0)),
        out_shape=jax.ShapeDtypeStruct((2, NPAD, D), jnp.float32),
    )(acc, g, dinv, W2, b1, a1)


def _final_body(acc_ref, g_ref, dinv_ref, b_ref, a_ref,
                pos_ref, neg_ref, sum_ref):
    i = pl.program_id(0)
    dv = dinv_ref[...]
    zp = dv * (acc_ref[0] + g_ref[0]) + b_ref[...]
    zp = jnp.where(zp >= 0, zp, a_ref[...] * zp)
    zn = dv * (acc_ref[1] + g_ref[1]) + b_ref[...]
    zn = jnp.where(zn >= 0, zn, a_ref[...] * zn)
    pos_ref[...] = zp
    neg_ref[...] = zn
    rows = lax.broadcasted_iota(jnp.int32, (BLK, 1), 0) + i * BLK
    part = jnp.sum(jnp.where(rows < N, zp, 0.0), axis=0, keepdims=True)

    @pl.when(i == 0)
    def _():
        sum_ref[...] = jnp.zeros_like(sum_ref)
    sum_ref[...] += part

    @pl.when(i == GRID - 1)
    def _():
        sum_ref[...] = jax.nn.sigmoid(sum_ref[...] * (1.0 / N))


def _final(acc, g, dinv, b2, a2):
    return pl.pallas_call(
        _final_body,
        grid=(GRID,),
        in_specs=[pl.BlockSpec((2, BLK, D), lambda i: (0, i, 0)),
                  pl.BlockSpec((2, BLK, D), lambda i: (0, i, 0)),
                  pl.BlockSpec((BLK, 1), lambda i: (i, 0)),
                  pl.BlockSpec((1, D), lambda i: (0, 0)),
                  pl.BlockSpec((1, D), lambda i: (0, 0))],
        out_specs=[pl.BlockSpec((BLK, D), lambda i: (i, 0)),
                   pl.BlockSpec((BLK, D), lambda i: (i, 0)),
                   pl.BlockSpec((1, D), lambda i: (0, 0))],
        out_shape=[jax.ShapeDtypeStruct((NPAD, D), jnp.float32),
                   jax.ShapeDtypeStruct((NPAD, D), jnp.float32),
                   jax.ShapeDtypeStruct((1, D), jnp.float32)],
    )(acc, g, dinv, b2, a2)


def kernel(x, edge_index, perm, W1, b1, a1, W2, b2, a2):
    x = x.astype(jnp.float32)
    srcp = edge_index[0].astype(jnp.int32)
    dstp = edge_index[1].astype(jnp.int32)
    permp = jnp.pad(perm.astype(jnp.int32), (0, NPAD - N))
    xp = jnp.pad(x, ((0, NPAD - N), (0, 0)))
    zrow = jnp.zeros((CH, D), jnp.float32)
    b1r = b1.reshape(1, D)
    a1r = a1.reshape(1, D)
    b2r = b2.reshape(1, D)
    a2r = a2.reshape(1, D)

    degp, xn = _stage_a(dstp, permp, xp)
    g1, dinv = _prep1(degp, xp, xn, W1)
    acc1 = _scatter(srcp, dstp, g1.reshape(NC * NPAD, D), zrow)
    g2 = _prep2(acc1.reshape(2, NPAD, D), g1, dinv, W2, b1r, a1r)
    acc2 = _scatter(srcp, dstp, g2.reshape(NC * NPAD, D), zrow)
    posz, negz, summ = _final(acc2.reshape(2, NPAD, D), g2, dinv, b2r, a2r)
    return posz[:N], negz[:N], summ.reshape(D)


# speedup vs baseline: 2.0983x; 1.0309x over previous
import jax
import jax.numpy as jnp
from jax import lax
from jax.experimental import pallas as pl
from jax.experimental.pallas import tpu as pltpu
from jax.experimental.pallas import tpu_sc as plsc

N = 10000
E = 320000
D = 128
NPAD = 10240
NC = 2
NS = 16
NW = NC * NS

BLK = 2048
GRID = NPAD // BLK

EPAD = E
EPW = EPAD // NW
EPT = EPAD // NS
CH = 80
SEG = 4000
NSEG = EPT // SEG
SEGC = SEG // CH
SEGP = SEGC // 2
RPT = NPAD // NS
RPW = NPAD // NW
CHA = 64


def _mesh():
    return plsc.VectorSubcoreMesh(
        core_axis_name="c", subcore_axis_name="s",
        num_cores=NC, num_subcores=NS)


def _stage_a_body(dst_hbm, perm_hbm, x_hbm, degp_hbm, xn_hbm,
                  cnt_v, idx_v, permb, rows_v, sem):
    c = lax.axis_index("c")
    s = lax.axis_index("s")
    w = s * NC + c

    zeros16 = jnp.zeros((16,), jnp.float32)

    def zbody(i, _):
        cnt_v[pl.ds(pl.multiple_of(i * 16, 16), 16)] = zeros16
        return 0
    lax.fori_loop(0, NPAD // 16, zbody, 0)

    pltpu.sync_copy(dst_hbm.at[pl.ds(pl.multiple_of(w * EPW, 8), EPW)], idx_v)

    ones16 = jnp.ones((16,), jnp.float32)

    def cbody(i, _):
        idx = idx_v[pl.ds(pl.multiple_of(i * 16, 16), 16)]
        plsc.addupdate_scatter(cnt_v, [idx], ones16)
        return 0
    lax.fori_loop(0, EPW // 16, cbody, 0)

    pltpu.sync_copy(cnt_v, degp_hbm.at[w])

    for k in range(RPW // CHA):
        base = w * RPW + k * CHA
        pltpu.sync_copy(perm_hbm.at[pl.ds(pl.multiple_of(base, 8), CHA)], permb)
        pltpu.async_copy(x_hbm.at[permb], rows_v, sem).wait()
        pltpu.sync_copy(rows_v, xn_hbm.at[pl.ds(pl.multiple_of(base, 8), CHA)])


def _stage_a(dstp, permp, xp):
    f = pl.kernel(
        _stage_a_body,
        out_type=(jax.ShapeDtypeStruct((NW, NPAD), jnp.float32),
                  jax.ShapeDtypeStruct((NPAD, D), jnp.float32)),
        mesh=_mesh(),
        scratch_types=[
            pltpu.VMEM((NPAD,), jnp.float32),
            pltpu.VMEM((EPW,), jnp.int32),
            pltpu.VMEM((CHA,), jnp.int32),
            pltpu.VMEM((CHA, D), jnp.float32),
            pltpu.SemaphoreType.DMA,
        ],
        compiler_params=pltpu.CompilerParams(needs_layout_passes=False),
    )
    return f(dstp, permp, xp)


def _scatter_body(src_hbm, dst_hbm, g_hbm, zrow_hbm, out_hbm,
                  srcall, dstall, srcall2, dstall2, dstb0, dstb1,
                  rows0, rows1, acc_sh, sg0, sg1, sd0, sd1):
    c = lax.axis_index("c")
    s = lax.axis_index("s")
    goff = c * NPAD

    pltpu.sync_copy(zrow_hbm, rows0)
    for k in range(RPT // CH):
        pltpu.async_copy(
            rows0, acc_sh.at[pl.ds(pl.multiple_of(s * RPT + k * CH, 8), CH)],
            sd0)
    for k in range(RPT // CH):
        pltpu.make_async_copy(
            rows0, acc_sh.at[pl.ds(pl.multiple_of(s * RPT + k * CH, 8), CH)],
            sd0).wait()
    plsc.subcore_barrier()

    def scat(db, rows):
        pltpu.sync_copy(rows, acc_sh.at[db], add=True)

    def seg_refs(seg):
        return (srcall, dstall) if seg % 2 == 0 else (srcall2, dstall2)

    def fire_seg(seg, sem):
        eb = pl.multiple_of(s * EPT + seg * SEG, 8)
        sa, da = seg_refs(seg)
        pltpu.async_copy(src_hbm.at[pl.ds(eb, SEG)], sa, sem)
        pltpu.async_copy(dst_hbm.at[pl.ds(eb, SEG)], da, sem)

    def wait_seg(seg, sem):
        eb = pl.multiple_of(s * EPT + seg * SEG, 8)
        sa, da = seg_refs(seg)
        pltpu.make_async_copy(src_hbm.at[pl.ds(eb, SEG)], sa, sem).wait()
        pltpu.make_async_copy(dst_hbm.at[pl.ds(eb, SEG)], da, sem).wait()

    fire_seg(0, sd0)
    wait_seg(0, sd0)

    for seg in range(NSEG):
        srca, dsta = seg_refs(seg)
        if seg + 1 < NSEG:
            fire_seg(seg + 1, sd1)

        def addoff(i, _):
            o = pl.multiple_of(i * 16, 16)
            srca[pl.ds(o, 16)] = srca[pl.ds(o, 16)] + goff
            return 0
        lax.fori_loop(0, SEG // 16, addoff, 0)

        def stage_dst_s(j, db):
            for t in range(CH // 16):
                o = pl.multiple_of(j * CH + t * 16, 16)
                db[pl.ds(t * 16, 16)] = dsta[pl.ds(o, 16)]

        def fire_g_s(j, rows, sem):
            pltpu.async_copy(
                g_hbm.at[srca.at[pl.ds(pl.multiple_of(j * CH, 8), CH)]],
                rows, sem)

        def wait_g_s(j, rows, sem):
            pltpu.make_async_copy(
                g_hbm.at[srca.at[pl.ds(pl.multiple_of(j * CH, 8), CH)]],
                rows, sem).wait()

        stage_dst_s(0, dstb0)
        fire_g_s(0, rows0, sg0)

        def pair(jj, _):
            j0 = jj * 2
            stage_dst_s(j0 + 1, dstb1)
            fire_g_s(j0 + 1, rows1, sg1)
            wait_g_s(j0, rows0, sg0)
            scat(dstb0, rows0)

            @pl.when(jj < SEGP - 1)
            def _():
                stage_dst_s(j0 + 2, dstb0)
                fire_g_s(j0 + 2, rows0, sg0)

            wait_g_s(j0 + 1, rows1, sg1)
            scat(dstb1, rows1)
            return 0
        lax.fori_loop(0, SEGP, pair, 0)

        if seg + 1 < NSEG:
            wait_seg(seg + 1, sd1)

    plsc.subcore_barrier()
    drn = RPT // CH
    for k in range(drn):
        rr = pl.multiple_of(s * RPT + k * CH, 8)
        rows, sem = (rows0, sd0) if k % 2 == 0 else (rows1, sd1)
        pltpu.sync_copy(acc_sh.at[pl.ds(rr, CH)], rows)
        pltpu.async_copy(rows, out_hbm.at[pl.ds(goff + rr, CH)], sem)
        if k >= 1:
            rp = pl.multiple_of(s * RPT + (k - 1) * CH, 8)
            rowsp, semp = (rows0, sd0) if (k - 1) % 2 == 0 else (rows1, sd1)
            pltpu.make_async_copy(
                rowsp, out_hbm.at[pl.ds(goff + rp, CH)], semp).wait()
    rl = pl.multiple_of(s * RPT + (drn - 1) * CH, 8)
    rowsl, seml = (rows0, sd0) if (drn - 1) % 2 == 0 else (rows1, sd1)
    pltpu.make_async_copy(
        rowsl, out_hbm.at[pl.ds(goff + rl, CH)], seml).wait()


def _scatter(srcp, dstp, gflat, zrow):
    f = pl.kernel(
        _scatter_body,
        out_type=jax.ShapeDtypeStruct((NC * NPAD, D), jnp.float32),
        mesh=_mesh(),
        scratch_types=[
            pltpu.VMEM((SEG,), jnp.int32),
            pltpu.VMEM((SEG,), jnp.int32),
            pltpu.VMEM((SEG,), jnp.int32),
            pltpu.VMEM((SEG,), jnp.int32),
            pltpu.VMEM((CH,), jnp.int32),
            pltpu.VMEM((CH,), jnp.int32),
            pltpu.VMEM((CH, D), jnp.float32),
            pltpu.VMEM((CH, D), jnp.float32),
            pltpu.VMEM_SHARED((NPAD, D), jnp.float32),
            pltpu.SemaphoreType.DMA,
            pltpu.SemaphoreType.DMA,
            pltpu.SemaphoreType.DMA,
            pltpu.SemaphoreType.DMA,
        ],
        compiler_params=pltpu.CompilerParams(needs_layout_passes=False),
    )
    return f(srcp, dstp, gflat, zrow)


def _prep1_body(degp_ref, xp_ref, xn_ref, w1_ref, g_ref, dinv_ref):
    deg = jnp.sum(degp_ref[...], axis=0) + 1.0
    dv = lax.rsqrt(deg)[:, None]
    dinv_ref[...] = dv
    g_ref[0] = dv * jnp.dot(xp_ref[...], w1_ref[...],
                            preferred_element_type=jnp.float32)
    g_ref[1] = dv * jnp.dot(xn_ref[...], w1_ref[...],
                            preferred_element_type=jnp.float32)


def _prep1(degp, xp, xn, W1):
    return pl.pallas_call(
        _prep1_body,
        grid=(GRID,),
        in_specs=[pl.BlockSpec((NW, BLK), lambda i: (0, i)),
                  pl.BlockSpec((BLK, D), lambda i: (i, 0)),
                  pl.BlockSpec((BLK, D), lambda i: (i, 0)),
                  pl.BlockSpec((D, D), lambda i: (0, 0))],
        out_specs=[pl.BlockSpec((2, BLK, D), lambda i: (0, i, 0)),
                   pl.BlockSpec((BLK, 1), lambda i: (i, 0))],
        out_shape=[jax.ShapeDtypeStruct((2, NPAD, D), jnp.float32),
                   jax.ShapeDtypeStruct((NPAD, 1), jnp.float32)],
    )(degp, xp, xn, W1)


def _prep2_body(acc_ref, g_ref, dinv_ref, w2_ref, b_ref, a_ref, o_ref):
    dv = dinv_ref[...]
    for k in range(2):
        z = dv * (acc_ref[k] + g_ref[k]) + b_ref[...]
        z = jnp.where(z >= 0, z, a_ref[...] * z)
        o_ref[k] = dv * jnp.dot(z, w2_ref[...],
                                preferred_element_type=jnp.float32)


def _prep2(acc, g, dinv, W2, b1, a1):
    return pl.pallas_call(
        _prep2_body,
        grid=(GRID,),
        in_specs=[pl.BlockSpec((2, BLK, D), lambda i: (0, i, 0)),
                  pl.BlockSpec((2, BLK, D), lambda i: (0, i, 0)),
                  pl.BlockSpec((BLK, 1), lambda i: (i, 0)),
                  pl.BlockSpec((D, D), lambda i: (0, 0)),
                  pl.BlockSpec((1, D), lambda i: (0, 0)),
                  pl.BlockSpec((1, D), lambda i: (0, 0))],
        out_specs=pl.BlockSpec((2, BLK, D), lambda i: (0, i, 0)),
        out_shape=jax.ShapeDtypeStruct((2, NPAD, D), jnp.float32),
    )(acc, g, dinv, W2, b1, a1)


def _final_body(acc_ref, g_ref, dinv_ref, b_ref, a_ref,
                pos_ref, neg_ref, sum_ref):
    i = pl.program_id(0)
    dv = dinv_ref[...]
    zp = dv * (acc_ref[0] + g_ref[0]) + b_ref[...]
    zp = jnp.where(zp >= 0, zp, a_ref[...] * zp)
    zn = dv * (acc_ref[1] + g_ref[1]) + b_ref[...]
    zn = jnp.where(zn >= 0, zn, a_ref[...] * zn)
    pos_ref[...] = zp
    neg_ref[...] = zn
    rows = lax.broadcasted_iota(jnp.int32, (BLK, 1), 0) + i * BLK
    part = jnp.sum(jnp.where(rows < N, zp, 0.0), axis=0, keepdims=True)

    @pl.when(i == 0)
    def _():
        sum_ref[...] = jnp.zeros_like(sum_ref)
    sum_ref[...] += part

    @pl.when(i == GRID - 1)
    def _():
        sum_ref[...] = jax.nn.sigmoid(sum_ref[...] * (1.0 / N))


def _final(acc, g, dinv, b2, a2):
    return pl.pallas_call(
        _final_body,
        grid=(GRID,),
        in_specs=[pl.BlockSpec((2, BLK, D), lambda i: (0, i, 0)),
                  pl.BlockSpec((2, BLK, D), lambda i: (0, i, 0)),
                  pl.BlockSpec((BLK, 1), lambda i: (i, 0)),
                  pl.BlockSpec((1, D), lambda i: (0, 0)),
                  pl.BlockSpec((1, D), lambda i: (0, 0))],
        out_specs=[pl.BlockSpec((BLK, D), lambda i: (i, 0)),
                   pl.BlockSpec((BLK, D), lambda i: (i, 0)),
                   pl.BlockSpec((1, D), lambda i: (0, 0))],
        out_shape=[jax.ShapeDtypeStruct((N, D), jnp.float32),
                   jax.ShapeDtypeStruct((N, D), jnp.float32),
                   jax.ShapeDtypeStruct((1, D), jnp.float32)],
    )(acc, g, dinv, b2, a2)


def kernel(x, edge_index, perm, W1, b1, a1, W2, b2, a2):
    x = x.astype(jnp.float32)
    srcp = edge_index[0].astype(jnp.int32)
    dstp = edge_index[1].astype(jnp.int32)
    permp = jnp.pad(perm.astype(jnp.int32), (0, NPAD - N))
    xp = jnp.pad(x, ((0, NPAD - N), (0, 0)))
    zrow = jnp.zeros((CH, D), jnp.float32)
    b1r = b1.reshape(1, D)
    a1r = a1.reshape(1, D)
    b2r = b2.reshape(1, D)
    a2r = a2.reshape(1, D)

    degp, xn = _stage_a(dstp, permp, xp)
    g1, dinv = _prep1(degp, xp, xn, W1)
    acc1 = _scatter(srcp, dstp, g1.reshape(NC * NPAD, D), zrow)
    g2 = _prep2(acc1.reshape(2, NPAD, D), g1, dinv, W2, b1r, a1r)
    acc2 = _scatter(srcp, dstp, g2.reshape(NC * NPAD, D), zrow)
    posz, negz, summ = _final(acc2.reshape(2, NPAD, D), g2, dinv, b2r, a2r)
    return posz, negz, summ.reshape(D)


# speedup vs baseline: 2.0986x; 1.0001x over previous
import jax
import jax.numpy as jnp
from jax import lax
from jax.experimental import pallas as pl
from jax.experimental.pallas import tpu as pltpu
from jax.experimental.pallas import tpu_sc as plsc

N = 10000
E = 320000
D = 128
NPAD = 10240
NC = 2
NS = 16
NW = NC * NS

BLK = 2048
GRID = NPAD // BLK

EPAD = E
EPW = EPAD // NW
EPT = EPAD // NS
CH = 80
SEG = 4000
NSEG = EPT // SEG
SEGC = SEG // CH
SEGP = SEGC // 2
RPT = NPAD // NS
RPW = NPAD // NW
CHA = 64


def _mesh():
    return plsc.VectorSubcoreMesh(
        core_axis_name="c", subcore_axis_name="s",
        num_cores=NC, num_subcores=NS)


def _stage_a_body(dst_hbm, perm_hbm, x_hbm, degp_hbm, xn_hbm,
                  cnt_v, idx_v, permb, rows_v, sem):
    c = lax.axis_index("c")
    s = lax.axis_index("s")
    w = s * NC + c

    zeros16 = jnp.zeros((16,), jnp.float32)

    def zbody(i, _):
        cnt_v[pl.ds(pl.multiple_of(i * 16, 16), 16)] = zeros16
        return 0
    lax.fori_loop(0, NPAD // 16, zbody, 0)

    pltpu.sync_copy(dst_hbm.at[pl.ds(pl.multiple_of(w * EPW, 8), EPW)], idx_v)

    ones16 = jnp.ones((16,), jnp.float32)

    def cbody(i, _):
        idx = idx_v[pl.ds(pl.multiple_of(i * 16, 16), 16)]
        plsc.addupdate_scatter(cnt_v, [idx], ones16)
        return 0
    lax.fori_loop(0, EPW // 16, cbody, 0)

    pltpu.sync_copy(cnt_v, degp_hbm.at[w])

    for k in range(RPW // CHA):
        base = w * RPW + k * CHA
        pltpu.sync_copy(perm_hbm.at[pl.ds(pl.multiple_of(base, 8), CHA)], permb)
        pltpu.async_copy(x_hbm.at[permb], rows_v, sem).wait()
        pltpu.sync_copy(rows_v, xn_hbm.at[pl.ds(pl.multiple_of(base, 8), CHA)])


def _stage_a(dstp, permp, xp):
    f = pl.kernel(
        _stage_a_body,
        out_type=(jax.ShapeDtypeStruct((NW, NPAD), jnp.float32),
                  jax.ShapeDtypeStruct((NPAD, D), jnp.float32)),
        mesh=_mesh(),
        scratch_types=[
            pltpu.VMEM((NPAD,), jnp.float32),
            pltpu.VMEM((EPW,), jnp.int32),
            pltpu.VMEM((CHA,), jnp.int32),
            pltpu.VMEM((CHA, D), jnp.float32),
            pltpu.SemaphoreType.DMA,
        ],
        compiler_params=pltpu.CompilerParams(needs_layout_passes=False),
    )
    return f(dstp, permp, xp)


def _scatter_body(src_hbm, dst_hbm, g_hbm, zrow_hbm, out_hbm,
                  srcall, dstall, srcall2, dstall2, dstb0, dstb1,
                  rows0, rows1, acc_sh, sg0, sg1, sd0, sd1):
    c = lax.axis_index("c")
    s = lax.axis_index("s")
    goff = c * NPAD

    pltpu.sync_copy(zrow_hbm, rows0)
    for k in range(RPT // CH):
        pltpu.async_copy(
            rows0, acc_sh.at[pl.ds(pl.multiple_of(s * RPT + k * CH, 8), CH)],
            sd0)
    for k in range(RPT // CH):
        pltpu.make_async_copy(
            rows0, acc_sh.at[pl.ds(pl.multiple_of(s * RPT + k * CH, 8), CH)],
            sd0).wait()
    plsc.subcore_barrier()

    def scat(db, rows):
        pltpu.sync_copy(rows, acc_sh.at[db], add=True)

    def seg_refs(seg):
        return (srcall, dstall) if seg % 2 == 0 else (srcall2, dstall2)

    def fire_seg(seg, sem):
        eb = pl.multiple_of(s * EPT + seg * SEG, 8)
        sa, da = seg_refs(seg)
        pltpu.async_copy(src_hbm.at[pl.ds(eb, SEG)], sa, sem)
        pltpu.async_copy(dst_hbm.at[pl.ds(eb, SEG)], da, sem)

    def wait_seg(seg, sem):
        eb = pl.multiple_of(s * EPT + seg * SEG, 8)
        sa, da = seg_refs(seg)
        pltpu.make_async_copy(src_hbm.at[pl.ds(eb, SEG)], sa, sem).wait()
        pltpu.make_async_copy(dst_hbm.at[pl.ds(eb, SEG)], da, sem).wait()

    fire_seg(0, sd0)
    wait_seg(0, sd0)

    for seg in range(NSEG):
        srca, dsta = seg_refs(seg)
        if seg + 1 < NSEG:
            fire_seg(seg + 1, sd1)

        def addoff(i, _):
            o = pl.multiple_of(i * 16, 16)
            srca[pl.ds(o, 16)] = srca[pl.ds(o, 16)] + goff
            return 0
        lax.fori_loop(0, SEG // 16, addoff, 0)

        def stage_dst_s(j, db):
            for t in range(CH // 16):
                o = pl.multiple_of(j * CH + t * 16, 16)
                db[pl.ds(t * 16, 16)] = dsta[pl.ds(o, 16)]

        def fire_g_s(j, rows, sem):
            pltpu.async_copy(
                g_hbm.at[srca.at[pl.ds(pl.multiple_of(j * CH, 8), CH)]],
                rows, sem)

        def wait_g_s(j, rows, sem):
            pltpu.make_async_copy(
                g_hbm.at[srca.at[pl.ds(pl.multiple_of(j * CH, 8), CH)]],
                rows, sem).wait()

        stage_dst_s(0, dstb0)
        fire_g_s(0, rows0, sg0)

        def pair(jj, _):
            j0 = jj * 2
            stage_dst_s(j0 + 1, dstb1)
            fire_g_s(j0 + 1, rows1, sg1)
            wait_g_s(j0, rows0, sg0)
            scat(dstb0, rows0)

            @pl.when(jj < SEGP - 1)
            def _():
                stage_dst_s(j0 + 2, dstb0)
                fire_g_s(j0 + 2, rows0, sg0)

            wait_g_s(j0 + 1, rows1, sg1)
            scat(dstb1, rows1)
            return 0
        lax.fori_loop(0, SEGP, pair, 0)

        if seg + 1 < NSEG:
            wait_seg(seg + 1, sd1)

    plsc.subcore_barrier()
    drn = RPT // CH
    for k in range(drn):
        rr = pl.multiple_of(s * RPT + k * CH, 8)
        rows, sem = (rows0, sd0) if k % 2 == 0 else (rows1, sd1)
        pltpu.sync_copy(acc_sh.at[pl.ds(rr, CH)], rows)
        pltpu.async_copy(rows, out_hbm.at[pl.ds(goff + rr, CH)], sem)
        if k >= 1:
            rp = pl.multiple_of(s * RPT + (k - 1) * CH, 8)
            rowsp, semp = (rows0, sd0) if (k - 1) % 2 == 0 else (rows1, sd1)
            pltpu.make_async_copy(
                rowsp, out_hbm.at[pl.ds(goff + rp, CH)], semp).wait()
    rl = pl.multiple_of(s * RPT + (drn - 1) * CH, 8)
    rowsl, seml = (rows0, sd0) if (drn - 1) % 2 == 0 else (rows1, sd1)
    pltpu.make_async_copy(
        rowsl, out_hbm.at[pl.ds(goff + rl, CH)], seml).wait()


def _scatter(srcp, dstp, gflat, zrow):
    f = pl.kernel(
        _scatter_body,
        out_type=jax.ShapeDtypeStruct((NC * NPAD, D), jnp.float32),
        mesh=_mesh(),
        scratch_types=[
            pltpu.VMEM((SEG,), jnp.int32),
            pltpu.VMEM((SEG,), jnp.int32),
            pltpu.VMEM((SEG,), jnp.int32),
            pltpu.VMEM((SEG,), jnp.int32),
            pltpu.VMEM((CH,), jnp.int32),
            pltpu.VMEM((CH,), jnp.int32),
            pltpu.VMEM((CH, D), jnp.float32),
            pltpu.VMEM((CH, D), jnp.float32),
            pltpu.VMEM_SHARED((NPAD, D), jnp.float32),
            pltpu.SemaphoreType.DMA,
            pltpu.SemaphoreType.DMA,
            pltpu.SemaphoreType.DMA,
            pltpu.SemaphoreType.DMA,
        ],
        compiler_params=pltpu.CompilerParams(needs_layout_passes=False),
    )
    return f(srcp, dstp, gflat, zrow)


def _mm_body(x_ref, w_ref, o_ref):
    o_ref[...] = jnp.dot(x_ref[...], w_ref[...],
                         preferred_element_type=jnp.float32)


def _mm(xp, W):
    return pl.pallas_call(
        _mm_body,
        grid=(GRID,),
        in_specs=[pl.BlockSpec((BLK, D), lambda i: (i, 0)),
                  pl.BlockSpec((D, D), lambda i: (0, 0))],
        out_specs=pl.BlockSpec((BLK, D), lambda i: (i, 0)),
        out_shape=jax.ShapeDtypeStruct((NPAD, D), jnp.float32),
    )(xp, W)


def _prep1_body(degp_ref, h1_ref, xn_ref, w1_ref, g_ref, dinv_ref):
    deg = jnp.sum(degp_ref[...], axis=0) + 1.0
    dv = lax.rsqrt(deg)[:, None]
    dinv_ref[...] = dv
    g_ref[0] = dv * h1_ref[...]
    g_ref[1] = dv * jnp.dot(xn_ref[...], w1_ref[...],
                            preferred_element_type=jnp.float32)


def _prep1(degp, h1, xn, W1):
    return pl.pallas_call(
        _prep1_body,
        grid=(GRID,),
        in_specs=[pl.BlockSpec((NW, BLK), lambda i: (0, i)),
                  pl.BlockSpec((BLK, D), lambda i: (i, 0)),
                  pl.BlockSpec((BLK, D), lambda i: (i, 0)),
                  pl.BlockSpec((D, D), lambda i: (0, 0))],
        out_specs=[pl.BlockSpec((2, BLK, D), lambda i: (0, i, 0)),
                   pl.BlockSpec((BLK, 1), lambda i: (i, 0))],
        out_shape=[jax.ShapeDtypeStruct((2, NPAD, D), jnp.float32),
                   jax.ShapeDtypeStruct((NPAD, 1), jnp.float32)],
    )(degp, h1, xn, W1)


def _prep2_body(acc_ref, g_ref, dinv_ref, w2_ref, b_ref, a_ref, o_ref):
    dv = dinv_ref[...]
    for k in range(2):
        z = dv * (acc_ref[k] + g_ref[k]) + b_ref[...]
        z = jnp.where(z >= 0, z, a_ref[...] * z)
        o_ref[k] = dv * jnp.dot(z, w2_ref[...],
                                preferred_element_type=jnp.float32)


def _prep2(acc, g, dinv, W2, b1, a1):
    return pl.pallas_call(
        _prep2_body,
        grid=(GRID,),
        in_specs=[pl.BlockSpec((2, BLK, D), lambda i: (0, i, 0)),
                  pl.BlockSpec((2, BLK, D), lambda i: (0, i, 0)),
                  pl.BlockSpec((BLK, 1), lambda i: (i, 0)),
                  pl.BlockSpec((D, D), lambda i: (0, 0)),
                  pl.BlockSpec((1, D), lambda i: (0, 0)),
                  pl.BlockSpec((1, D), lambda i: (0, 0))],
        out_specs=pl.BlockSpec((2, BLK, D), lambda i: (0, i, 0)),
        out_shape=jax.ShapeDtypeStruct((2, NPAD, D), jnp.float32),
    )(acc, g, dinv, W2, b1, a1)


def _final_body(acc_ref, g_ref, dinv_ref, b_ref, a_ref,
                pos_ref, neg_ref, sum_ref):
    i = pl.program_id(0)
    dv = dinv_ref[...]
    zp = dv * (acc_ref[0] + g_ref[0]) + b_ref[...]
    zp = jnp.where(zp >= 0, zp, a_ref[...] * zp)
    zn = dv * (acc_ref[1] + g_ref[1]) + b_ref[...]
    zn = jnp.where(zn >= 0, zn, a_ref[...] * zn)
    pos_ref[...] = zp
    neg_ref[...] = zn
    rows = lax.broadcasted_iota(jnp.int32, (BLK, 1), 0) + i * BLK
    part = jnp.sum(jnp.where(rows < N, zp, 0.0), axis=0, keepdims=True)

    @pl.when(i == 0)
    def _():
        sum_ref[...] = jnp.zeros_like(sum_ref)
    sum_ref[...] += part

    @pl.when(i == GRID - 1)
    def _():
        sum_ref[...] = jax.nn.sigmoid(sum_ref[...] * (1.0 / N))


def _final(acc, g, dinv, b2, a2):
    return pl.pallas_call(
        _final_body,
        grid=(GRID,),
        in_specs=[pl.BlockSpec((2, BLK, D), lambda i: (0, i, 0)),
                  pl.BlockSpec((2, BLK, D), lambda i: (0, i, 0)),
                  pl.BlockSpec((BLK, 1), lambda i: (i, 0)),
                  pl.BlockSpec((1, D), lambda i: (0, 0)),
                  pl.BlockSpec((1, D), lambda i: (0, 0))],
        out_specs=[pl.BlockSpec((BLK, D), lambda i: (i, 0)),
                   pl.BlockSpec((BLK, D), lambda i: (i, 0)),
                   pl.BlockSpec((1, D), lambda i: (0, 0))],
        out_shape=[jax.ShapeDtypeStruct((N, D), jnp.float32),
                   jax.ShapeDtypeStruct((N, D), jnp.float32),
                   jax.ShapeDtypeStruct((1, D), jnp.float32)],
    )(acc, g, dinv, b2, a2)


def kernel(x, edge_index, perm, W1, b1, a1, W2, b2, a2):
    x = x.astype(jnp.float32)
    srcp = edge_index[0].astype(jnp.int32)
    dstp = edge_index[1].astype(jnp.int32)
    permp = jnp.pad(perm.astype(jnp.int32), (0, NPAD - N))
    xp = jnp.pad(x, ((0, NPAD - N), (0, 0)))
    zrow = jnp.zeros((CH, D), jnp.float32)
    b1r = b1.reshape(1, D)
    a1r = a1.reshape(1, D)
    b2r = b2.reshape(1, D)
    a2r = a2.reshape(1, D)

    h1 = _mm(xp, W1)
    degp, xn = _stage_a(dstp, permp, xp)
    g1, dinv = _prep1(degp, h1, xn, W1)
    acc1 = _scatter(srcp, dstp, g1.reshape(NC * NPAD, D), zrow)
    g2 = _prep2(acc1.reshape(2, NPAD, D), g1, dinv, W2, b1r, a1r)
    acc2 = _scatter(srcp, dstp, g2.reshape(NC * NPAD, D), zrow)
    posz, negz, summ = _final(acc2.reshape(2, NPAD, D), g2, dinv, b2r, a2r)
    return posz, negz, summ.reshape(D)


# speedup vs baseline: 2.1002x; 1.0008x over previous
import jax
import jax.numpy as jnp
from jax import lax
from jax.experimental import pallas as pl
from jax.experimental.pallas import tpu as pltpu
from jax.experimental.pallas import tpu_sc as plsc

N = 10000
E = 320000
D = 128
NPAD = 10240
NC = 2
NS = 16
NW = NC * NS

BLK = 2048
GRID = NPAD // BLK

EPAD = E
EPW = EPAD // NW
EPT = EPAD // NS
CH = 80
SEG = 4000
NSEG = EPT // SEG
SEGC = SEG // CH
SEGP = SEGC // 2
RPT = NPAD // NS
RPW = NPAD // NW
CHA = 64


def _mesh():
    return plsc.VectorSubcoreMesh(
        core_axis_name="c", subcore_axis_name="s",
        num_cores=NC, num_subcores=NS)


def _stage_a_body(dst_hbm, perm_hbm, x_hbm, degp_hbm, xn_hbm,
                  cnt_v, idx_v, permb, rows_v, sem):
    c = lax.axis_index("c")
    s = lax.axis_index("s")
    w = s * NC + c

    zeros16 = jnp.zeros((16,), jnp.float32)

    def zbody(i, _):
        cnt_v[pl.ds(pl.multiple_of(i * 16, 16), 16)] = zeros16
        return 0
    lax.fori_loop(0, NPAD // 16, zbody, 0)

    pltpu.sync_copy(dst_hbm.at[pl.ds(pl.multiple_of(w * EPW, 8), EPW)], idx_v)

    ones16 = jnp.ones((16,), jnp.float32)

    def cbody(i, _):
        idx = idx_v[pl.ds(pl.multiple_of(i * 16, 16), 16)]
        plsc.addupdate_scatter(cnt_v, [idx], ones16)
        return 0
    lax.fori_loop(0, EPW // 16, cbody, 0)

    pltpu.sync_copy(cnt_v, degp_hbm.at[w])

    for k in range(RPW // CHA):
        base = w * RPW + k * CHA
        pltpu.sync_copy(perm_hbm.at[pl.ds(pl.multiple_of(base, 8), CHA)], permb)
        pltpu.async_copy(x_hbm.at[permb], rows_v, sem).wait()
        pltpu.sync_copy(rows_v, xn_hbm.at[pl.ds(pl.multiple_of(base, 8), CHA)])


def _stage_a(dstp, permp, xp):
    f = pl.kernel(
        _stage_a_body,
        out_type=(jax.ShapeDtypeStruct((NW, NPAD), jnp.float32),
                  jax.ShapeDtypeStruct((NPAD, D), jnp.float32)),
        mesh=_mesh(),
        scratch_types=[
            pltpu.VMEM((NPAD,), jnp.float32),
            pltpu.VMEM((EPW,), jnp.int32),
            pltpu.VMEM((CHA,), jnp.int32),
            pltpu.VMEM((CHA, D), jnp.float32),
            pltpu.SemaphoreType.DMA,
        ],
        compiler_params=pltpu.CompilerParams(needs_layout_passes=False),
    )
    return f(dstp, permp, xp)


def _scatter_body(src_hbm, dst_hbm, g_hbm, zrow_hbm, out_hbm,
                  srcall, dstall, srcall2, dstall2, dstb0, dstb1,
                  rows0, rows1, acc_sh, sg0, sg1, sd0, sd1):
    c = lax.axis_index("c")
    s = lax.axis_index("s")
    goff = c * NPAD

    pltpu.sync_copy(zrow_hbm, rows0)
    for k in range(RPT // CH):
        pltpu.async_copy(
            rows0, acc_sh.at[pl.ds(pl.multiple_of(s * RPT + k * CH, 8), CH)],
            sd0)
    for k in range(RPT // CH):
        pltpu.make_async_copy(
            rows0, acc_sh.at[pl.ds(pl.multiple_of(s * RPT + k * CH, 8), CH)],
            sd0).wait()
    plsc.subcore_barrier()

    def scat(db, rows):
        pltpu.sync_copy(rows, acc_sh.at[db], add=True)

    def seg_refs(seg):
        return (srcall, dstall) if seg % 2 == 0 else (srcall2, dstall2)

    def fire_seg(seg, sem):
        eb = pl.multiple_of(s * EPT + seg * SEG, 8)
        sa, da = seg_refs(seg)
        pltpu.async_copy(src_hbm.at[pl.ds(eb, SEG)], sa, sem)
        pltpu.async_copy(dst_hbm.at[pl.ds(eb, SEG)], da, sem)

    def wait_seg(seg, sem):
        eb = pl.multiple_of(s * EPT + seg * SEG, 8)
        sa, da = seg_refs(seg)
        pltpu.make_async_copy(src_hbm.at[pl.ds(eb, SEG)], sa, sem).wait()
        pltpu.make_async_copy(dst_hbm.at[pl.ds(eb, SEG)], da, sem).wait()

    fire_seg(0, sd0)
    wait_seg(0, sd0)

    for seg in range(NSEG):
        srca, dsta = seg_refs(seg)
        if seg + 1 < NSEG:
            fire_seg(seg + 1, sd1)

        def addoff(i, _):
            o = pl.multiple_of(i * 16, 16)
            srca[pl.ds(o, 16)] = srca[pl.ds(o, 16)] + goff
            return 0
        lax.fori_loop(0, SEG // 16, addoff, 0)

        def stage_dst_s(j, db):
            for t in range(CH // 16):
                o = pl.multiple_of(j * CH + t * 16, 16)
                db[pl.ds(t * 16, 16)] = dsta[pl.ds(o, 16)]

        def fire_g_s(j, rows, sem):
            pltpu.async_copy(
                g_hbm.at[srca.at[pl.ds(pl.multiple_of(j * CH, 8), CH)]],
                rows, sem)

        def wait_g_s(j, rows, sem):
            pltpu.make_async_copy(
                g_hbm.at[srca.at[pl.ds(pl.multiple_of(j * CH, 8), CH)]],
                rows, sem).wait()

        stage_dst_s(0, dstb0)
        fire_g_s(0, rows0, sg0)

        def pair(jj, _):
            j0 = jj * 2
            stage_dst_s(j0 + 1, dstb1)
            fire_g_s(j0 + 1, rows1, sg1)
            wait_g_s(j0, rows0, sg0)
            scat(dstb0, rows0)

            @pl.when(jj < SEGP - 1)
            def _():
                stage_dst_s(j0 + 2, dstb0)
                fire_g_s(j0 + 2, rows0, sg0)

            wait_g_s(j0 + 1, rows1, sg1)
            scat(dstb1, rows1)
            return 0
        lax.fori_loop(0, SEGP, pair, 0)

        if seg + 1 < NSEG:
            wait_seg(seg + 1, sd1)

    plsc.subcore_barrier()
    drn = RPT // CH
    for k in range(drn):
        rr = pl.multiple_of(s * RPT + k * CH, 8)
        rows, sem = (rows0, sd0) if k % 2 == 0 else (rows1, sd1)
        pltpu.sync_copy(acc_sh.at[pl.ds(rr, CH)], rows)
        pltpu.async_copy(rows, out_hbm.at[pl.ds(goff + rr, CH)], sem)
        if k >= 1:
            rp = pl.multiple_of(s * RPT + (k - 1) * CH, 8)
            rowsp, semp = (rows0, sd0) if (k - 1) % 2 == 0 else (rows1, sd1)
            pltpu.make_async_copy(
                rowsp, out_hbm.at[pl.ds(goff + rp, CH)], semp).wait()
    rl = pl.multiple_of(s * RPT + (drn - 1) * CH, 8)
    rowsl, seml = (rows0, sd0) if (drn - 1) % 2 == 0 else (rows1, sd1)
    pltpu.make_async_copy(
        rowsl, out_hbm.at[pl.ds(goff + rl, CH)], seml).wait()


def _scatter(srcp, dstp, gflat, zrow):
    f = pl.kernel(
        _scatter_body,
        out_type=jax.ShapeDtypeStruct((NC * NPAD, D), jnp.float32),
        mesh=_mesh(),
        scratch_types=[
            pltpu.VMEM((SEG,), jnp.int32),
            pltpu.VMEM((SEG,), jnp.int32),
            pltpu.VMEM((SEG,), jnp.int32),
            pltpu.VMEM((SEG,), jnp.int32),
            pltpu.VMEM((CH,), jnp.int32),
            pltpu.VMEM((CH,), jnp.int32),
            pltpu.VMEM((CH, D), jnp.float32),
            pltpu.VMEM((CH, D), jnp.float32),
            pltpu.VMEM_SHARED((NPAD, D), jnp.float32),
            pltpu.SemaphoreType.DMA,
            pltpu.SemaphoreType.DMA,
            pltpu.SemaphoreType.DMA,
            pltpu.SemaphoreType.DMA,
        ],
        compiler_params=pltpu.CompilerParams(needs_layout_passes=False),
    )
    return f(srcp, dstp, gflat, zrow)


def _prep1_body(degp_ref, xp_ref, xn_ref, w1_ref, g_ref, dinv_ref):
    deg = jnp.sum(degp_ref[...], axis=0) + 1.0
    dv = lax.rsqrt(deg)[:, None]
    dinv_ref[...] = dv
    g_ref[0] = dv * jnp.dot(xp_ref[...], w1_ref[...],
                            preferred_element_type=jnp.float32)
    g_ref[1] = dv * jnp.dot(xn_ref[...], w1_ref[...],
                            preferred_element_type=jnp.float32)


def _prep1(degp, xp, xn, W1):
    return pl.pallas_call(
        _prep1_body,
        grid=(GRID,),
        in_specs=[pl.BlockSpec((NW, BLK), lambda i: (0, i)),
                  pl.BlockSpec((BLK, D), lambda i: (i, 0)),
                  pl.BlockSpec((BLK, D), lambda i: (i, 0)),
                  pl.BlockSpec((D, D), lambda i: (0, 0))],
        out_specs=[pl.BlockSpec((2, BLK, D), lambda i: (0, i, 0)),
                   pl.BlockSpec((BLK, 1), lambda i: (i, 0))],
        out_shape=[jax.ShapeDtypeStruct((2, NPAD, D), jnp.float32),
                   jax.ShapeDtypeStruct((NPAD, 1), jnp.float32)],
    )(degp, xp, xn, W1)


def _prep2_body(acc_ref, g_ref, dinv_ref, w2_ref, b_ref, a_ref, o_ref):
    dv = dinv_ref[...]
    for k in range(2):
        z = dv * (acc_ref[k] + g_ref[k]) + b_ref[...]
        z = jnp.where(z >= 0, z, a_ref[...] * z)
        o_ref[k] = dv * jnp.dot(z, w2_ref[...],
                                preferred_element_type=jnp.float32)


def _prep2(acc, g, dinv, W2, b1, a1):
    return pl.pallas_call(
        _prep2_body,
        grid=(GRID,),
        in_specs=[pl.BlockSpec((2, BLK, D), lambda i: (0, i, 0)),
                  pl.BlockSpec((2, BLK, D), lambda i: (0, i, 0)),
                  pl.BlockSpec((BLK, 1), lambda i: (i, 0)),
                  pl.BlockSpec((D, D), lambda i: (0, 0)),
                  pl.BlockSpec((1, D), lambda i: (0, 0)),
                  pl.BlockSpec((1, D), lambda i: (0, 0))],
        out_specs=pl.BlockSpec((2, BLK, D), lambda i: (0, i, 0)),
        out_shape=jax.ShapeDtypeStruct((2, NPAD, D), jnp.float32),
    )(acc, g, dinv, W2, b1, a1)


def _final_body(acc_ref, g_ref, dinv_ref, b_ref, a_ref,
                pos_ref, neg_ref, sum_ref):
    i = pl.program_id(0)
    dv = dinv_ref[...]
    zp = dv * (acc_ref[0] + g_ref[0]) + b_ref[...]
    zp = jnp.where(zp >= 0, zp, a_ref[...] * zp)
    zn = dv * (acc_ref[1] + g_ref[1]) + b_ref[...]
    zn = jnp.where(zn >= 0, zn, a_ref[...] * zn)
    pos_ref[...] = zp
    neg_ref[...] = zn
    rows = lax.broadcasted_iota(jnp.int32, (BLK, 1), 0) + i * BLK
    part = jnp.sum(jnp.where(rows < N, zp, 0.0), axis=0, keepdims=True)

    @pl.when(i == 0)
    def _():
        sum_ref[...] = jnp.zeros_like(sum_ref)
    sum_ref[...] += part

    @pl.when(i == GRID - 1)
    def _():
        sum_ref[...] = jax.nn.sigmoid(sum_ref[...] * (1.0 / N))


def _final(acc, g, dinv, b2, a2):
    return pl.pallas_call(
        _final_body,
        grid=(GRID,),
        in_specs=[pl.BlockSpec((2, BLK, D), lambda i: (0, i, 0)),
                  pl.BlockSpec((2, BLK, D), lambda i: (0, i, 0)),
                  pl.BlockSpec((BLK, 1), lambda i: (i, 0)),
                  pl.BlockSpec((1, D), lambda i: (0, 0)),
                  pl.BlockSpec((1, D), lambda i: (0, 0))],
        out_specs=[pl.BlockSpec((BLK, D), lambda i: (i, 0)),
                   pl.BlockSpec((BLK, D), lambda i: (i, 0)),
                   pl.BlockSpec((1, D), lambda i: (0, 0))],
        out_shape=[jax.ShapeDtypeStruct((N, D), jnp.float32),
                   jax.ShapeDtypeStruct((N, D), jnp.float32),
                   jax.ShapeDtypeStruct((1, D), jnp.float32)],
    )(acc, g, dinv, b2, a2)


def kernel(x, edge_index, perm, W1, b1, a1, W2, b2, a2):
    x = x.astype(jnp.float32)
    srcp = edge_index[0].astype(jnp.int32)
    dstp = edge_index[1].astype(jnp.int32)
    permp = jnp.pad(perm.astype(jnp.int32), (0, NPAD - N))
    xp = jnp.pad(x, ((0, NPAD - N), (0, 0)))
    zrow = jnp.zeros((CH, D), jnp.float32)
    b1r = b1.reshape(1, D)
    a1r = a1.reshape(1, D)
    b2r = b2.reshape(1, D)
    a2r = a2.reshape(1, D)

    degp, xn = _stage_a(dstp, permp, xp)
    g1, dinv = _prep1(degp, xp, xn, W1)
    acc1 = _scatter(srcp, dstp, g1.reshape(NC * NPAD, D), zrow)
    g2 = _prep2(acc1.reshape(2, NPAD, D), g1, dinv, W2, b1r, a1r)
    acc2 = _scatter(srcp, dstp, g2.reshape(NC * NPAD, D), zrow)
    posz, negz, summ = _final(acc2.reshape(2, NPAD, D), g2, dinv, b2r, a2r)
    return posz, negz, summ.reshape(D)
